# Initial kernel scaffold; baseline (speedup 1.0000x reference)
import jax
import jax.numpy as jnp
from jax.experimental import pallas as pl


def kernel(x, edge_index, batch, params):
    raise NotImplementedError("write your pallas kernel here")


# speedup vs baseline: 7.1863x; 7.1863x over previous
import functools

import jax
import jax.numpy as jnp
from jax import lax
from jax.experimental import pallas as pl
from jax.experimental.pallas import tpu as pltpu
from jax.experimental.pallas import tpu_sc as plsc

N = 10000
NPAD = 10240
H = 128
BLK = 512
NBLK = NPAD // BLK

NC = 2
NS = 16
EB = 128
KB = 160
EPAD = NS * KB * EB
RPT = NPAD // NS
HH = H // 2


def _ln(u, g, b):
    mu = jnp.mean(u, axis=-1, keepdims=True)
    var = jnp.mean((u - mu) ** 2, axis=-1, keepdims=True)
    return (u - mu) / jnp.sqrt(var + 1e-5) * g + b


def _store_ep(ep_ref, e, p):
    ep_ref[0, 0] = e[:, :HH]
    ep_ref[0, 1] = e[:, HH:]
    ep_ref[1, 0] = p[:, :HH]
    ep_ref[1, 1] = p[:, HH:]


def _agg_from_sw(sw_ref):
    return jnp.concatenate(
        [sw_ref[1, 0] / (sw_ref[0, 0] + 1e-16),
         sw_ref[1, 1] / (sw_ref[0, 1] + 1e-16)], axis=-1)


def _embed(xp, fc_w, fc_b, t0):
    def body(t_ref, x_ref, w_ref, b_ref, h_ref, ep_ref):
        h = jnp.maximum(
            jnp.dot(x_ref[...], w_ref[...], preferred_element_type=jnp.float32)
            + b_ref[...], 0.0)
        h_ref[...] = h
        m = h + 1e-7
        e = jnp.exp(t_ref[0, 0] * m)
        _store_ep(ep_ref, e, m * e)

    return pl.pallas_call(
        body,
        grid=(NBLK,),
        in_specs=[
            pl.BlockSpec(memory_space=pltpu.SMEM),
            pl.BlockSpec((BLK, H), lambda b: (b, 0)),
            pl.BlockSpec((H, H), lambda b: (0, 0)),
            pl.BlockSpec((1, H), lambda b: (0, 0)),
        ],
        out_specs=[
            pl.BlockSpec((BLK, H), lambda b: (b, 0)),
            pl.BlockSpec((2, 2, BLK, HH), lambda b: (0, 0, b, 0)),
        ],
        out_shape=[
            jax.ShapeDtypeStruct((NPAD, H), jnp.float32),
            jax.ShapeDtypeStruct((2, 2, NPAD, HH), jnp.float32),
        ],
    )(t0, xp, fc_w, fc_b.reshape(1, -1))


def _conv(h, sw, cp, t_next, residual):
    has_next = t_next is not None

    def body(*refs):
        if residual:
            (t_ref, h_ref, sw_ref, w1, b1, g1, be1, w2, b2, ng, nb) = refs[:11]
            outs = refs[11:]
        else:
            (t_ref, h_ref, sw_ref, w1, b1, g1, be1, w2, b2) = refs[:9]
            outs = refs[9:]
        h_out_ref = outs[0]
        agg = _agg_from_sw(sw_ref)
        h_in = h_ref[...]
        z = agg + h_in
        u = jnp.dot(z, w1[...], preferred_element_type=jnp.float32) + b1[...]
        u = jnp.maximum(_ln(u, g1[...], be1[...]), 0.0)
        v = jnp.dot(u, w2[...], preferred_element_type=jnp.float32) + b2[...]
        if residual:
            h_new = h_in + jnp.maximum(_ln(v, ng[...], nb[...]), 0.0)
        else:
            h_new = v
        h_out_ref[...] = h_new
        if has_next:
            ep_ref = outs[1]
            m = jnp.maximum(h_new, 0.0) + 1e-7
            e = jnp.exp(t_ref[0, 0] * m)
            _store_ep(ep_ref, e, m * e)

    row = lambda shp: pl.BlockSpec(shp, lambda b: (0, 0))
    in_specs = [
        pl.BlockSpec(memory_space=pltpu.SMEM),
        pl.BlockSpec((BLK, H), lambda b: (b, 0)),
        pl.BlockSpec((2, 2, BLK, HH), lambda b: (0, 0, b, 0)),
        row((H, 2 * H)), row((1, 2 * H)), row((1, 2 * H)), row((1, 2 * H)),
        row((2 * H, H)), row((1, H)),
    ]
    args = [
        jnp.ones((1, 1), jnp.float32) if t_next is None else t_next,
        h, sw,
        cp["w1"], cp["b1"].reshape(1, -1),
        cp["ln_g"].reshape(1, -1), cp["ln_b"].reshape(1, -1),
        cp["w2"], cp["b2"].reshape(1, -1),
    ]
    if residual:
        in_specs += [row((1, H)), row((1, H))]
        args += [cp["nrm_g"].reshape(1, -1), cp["nrm_b"].reshape(1, -1)]
    out_specs = [pl.BlockSpec((BLK, H), lambda b: (b, 0))]
    out_shape = [jax.ShapeDtypeStruct((NPAD, H), jnp.float32)]
    if has_next:
        out_specs.append(pl.BlockSpec((2, 2, BLK, HH), lambda b: (0, 0, b, 0)))
        out_shape.append(jax.ShapeDtypeStruct((2, 2, NPAD, HH), jnp.float32))

    res = pl.pallas_call(
        body, grid=(NBLK,),
        in_specs=in_specs, out_specs=out_specs, out_shape=out_shape,
    )(*args)
    return res if has_next else (res[0], None)


def _segsum(v4, srcb, dstb):
    mesh = plsc.VectorSubcoreMesh(core_axis_name="c", subcore_axis_name="s")

    @functools.partial(
        pl.kernel,
        out_type=jax.ShapeDtypeStruct((NC, 2, NPAD, HH), jnp.float32),
        mesh=mesh,
        compiler_params=pltpu.CompilerParams(use_tc_tiling_on_sc=False),
        scratch_types=[
            pltpu.VMEM((KB, EB), jnp.int32),
            pltpu.VMEM((KB, EB), jnp.int32),
            pltpu.VMEM((4, EB, HH), jnp.float32),
            pltpu.VMEM((EB, HH), jnp.float32),
            pltpu.VMEM_SHARED((NPAD, HH), jnp.float32),
            pltpu.SemaphoreType.DMA((4,)),
            pltpu.SemaphoreType.DMA((4,)),
        ],
    )
    def k(v4_hbm, srcb_hbm, dstb_hbm, out_hbm, src_v, dst_v, rows_v, zb_v,
          acc, gsem, ssem):
        c = lax.axis_index("c")
        s = lax.axis_index("s")
        pltpu.sync_copy(dstb_hbm.at[s], dst_v)

        zeros16 = jnp.zeros((16,), jnp.float32)

        def zrow(i, carry):
            for j in range(HH // 16):
                zb_v[i, pl.ds(j * 16, 16)] = zeros16
            return carry

        lax.fori_loop(0, EB, zrow, 0)

        def gather_cp(j, b):
            return pltpu.make_async_copy(
                v4_hbm.at[src_v.at[j]], rows_v.at[b], gsem.at[b])

        def scatter_cp(j, b):
            return pltpu.make_async_copy(
                rows_v.at[b], acc.at[dst_v.at[j]], ssem.at[b])

        for h in range(2):
            pltpu.sync_copy(srcb_hbm.at[c, h, s], src_v)
            for r in range(RPT // EB):
                pltpu.sync_copy(zb_v, acc.at[pl.ds(s * RPT + r * EB, EB)])
            plsc.subcore_barrier()

            gather_cp(0, 0).start()
            gather_cp(1, 1).start()

            def step(g, carry):
                for b2 in range(2):
                    j = g * 2 + b2
                    jb = j % 4
                    gather_cp(j, jb).wait()
                    scatter_cp(j, jb).start(add=True)

                    @pl.when(j + 2 < KB)
                    def _():
                        @pl.when(j >= 2)
                        def _():
                            scatter_cp(j - 2, (j - 2) % 4).wait()
                        gather_cp(j + 2, (j + 2) % 4).start()
                return carry

            lax.fori_loop(0, KB // 2, step, 0)
            for jt in range(KB - 4, KB):
                scatter_cp(jt, jt % 4).wait()
            plsc.subcore_barrier()
            pltpu.sync_copy(acc.at[pl.ds(s * RPT, RPT)],
                            out_hbm.at[c, h, pl.ds(s * RPT, RPT)])

    return k(v4, srcb, dstb)


def _attn(hs, p):
    def body(h0r, h1r, h2r, h3r, phw, phb, awr, abr, bwr, bbr, cwr, cbr,
             rwr, rbr, clw, clb, out_ref, vacc, msc, zsc):
        b = pl.program_id(0)

        @pl.when(b == 0)
        def _():
            msc[0, 0] = -1e30
            zsc[0, 0] = 0.0
            vacc[...] = jnp.zeros_like(vacc)

        hp = jnp.maximum(
            jnp.dot(h0r[...], phw[0:H], preferred_element_type=jnp.float32)
            + jnp.dot(h1r[...], phw[H:2 * H], preferred_element_type=jnp.float32)
            + jnp.dot(h2r[...], phw[2 * H:3 * H], preferred_element_type=jnp.float32)
            + jnp.dot(h3r[...], phw[3 * H:], preferred_element_type=jnp.float32)
            + phb[...], 0.0)
        a = jnp.tanh(jnp.dot(hp, awr[...], preferred_element_type=jnp.float32)
                     + abr[...])
        g = jax.nn.sigmoid(jnp.dot(hp, bwr[...], preferred_element_type=jnp.float32)
                           + bbr[...])
        sc = jnp.dot(a * g, cwr[...], preferred_element_type=jnp.float32) + cbr[...]
        rows = b * BLK + lax.broadcasted_iota(jnp.int32, (BLK, 1), 0)
        sc = jnp.where(rows < N, sc, -1e30)

        m_old = msc[0, 0]
        m_new = jnp.maximum(m_old, jnp.max(sc))
        scale = jnp.exp(m_old - m_new)
        e = jnp.exp(sc - m_new)
        zsc[0, 0] = zsc[0, 0] * scale + jnp.sum(e)
        vacc[...] = vacc[...] * scale + jnp.sum(hp * e, axis=0, keepdims=True)
        msc[0, 0] = m_new

        hpool = vacc[...] / zsc[0, 0]
        hres = jnp.maximum(
            jnp.dot(hpool, rwr[...], preferred_element_type=jnp.float32)
            + rbr[...], 0.0)
        logit = jnp.dot(hres, clw[...], preferred_element_type=jnp.float32) + clb[...]
        out_ref[...] = jax.nn.sigmoid(logit)

    D4 = 4 * H
    row = lambda shp: pl.BlockSpec(shp, lambda b: (0, 0))
    hblk = pl.BlockSpec((BLK, H), lambda b: (b, 0))
    return pl.pallas_call(
        body,
        grid=(NBLK,),
        in_specs=[hblk, hblk, hblk, hblk,
                  row((D4, D4)), row((1, D4)),
                  row((D4, D4)), row((1, D4)),
                  row((D4, D4)), row((1, D4)),
                  row((D4, 1)), row((1, 1)),
                  row((D4, D4)), row((1, D4)),
                  row((D4, 1)), row((1, 1))],
        out_specs=pl.BlockSpec((1, 1), lambda b: (0, 0)),
        out_shape=jax.ShapeDtypeStruct((1, 1), jnp.float32),
        scratch_shapes=[
            pltpu.VMEM((1, D4), jnp.float32),
            pltpu.SMEM((1, 1), jnp.float32),
            pltpu.SMEM((1, 1), jnp.float32),
        ],
    )(hs[0], hs[1], hs[2], hs[3],
      p["phi_w"], p["phi_b"].reshape(1, -1),
      p["aw"], p["ab"].reshape(1, -1),
      p["bw"], p["bb"].reshape(1, -1),
      p["cw"], p["cb"].reshape(1, 1),
      p["rho_w"], p["rho_b"].reshape(1, -1),
      p["cls_w"], p["cls_b"].reshape(1, 1))


def kernel(x, edge_index, batch, params):
    src = edge_index[0]
    dst = edge_index[1]
    pad = EPAD - src.shape[0]
    srcp = jnp.concatenate([src, jnp.full((pad,), N, jnp.int32)])
    dstp = jnp.concatenate([dst, jnp.full((pad,), N, jnp.int32)])
    srcb = (srcp[None, None] +
            (jnp.arange(4, dtype=jnp.int32) * NPAD).reshape(2, 2, 1)
            ).reshape(NC, 2, NS, KB, EB)
    dstb = dstp.reshape(NS, KB, EB)
    xp = jnp.pad(x, ((0, NPAD - N), (0, 0)))

    cs = params["convs"]
    t = [c["t"].reshape(1, 1) for c in cs]

    h0, ep0 = _embed(xp, params["fc_w"], params["fc_b"], t[0])
    sw0 = _segsum(ep0.reshape(4 * NPAD, HH), srcb, dstb)
    h1, ep1 = _conv(h0, sw0, cs[0], t[1], residual=False)
    sw1 = _segsum(ep1.reshape(4 * NPAD, HH), srcb, dstb)
    h2, ep2 = _conv(h1, sw1, cs[1], t[2], residual=True)
    sw2 = _segsum(ep2.reshape(4 * NPAD, HH), srcb, dstb)
    h3, _ = _conv(h2, sw2, cs[2], None, residual=True)
    return _attn((h0, h1, h2, h3), params)


# speedup vs baseline: 9.4398x; 1.3136x over previous
import functools

import jax
import jax.numpy as jnp
from jax import lax
from jax.experimental import pallas as pl
from jax.experimental.pallas import tpu as pltpu
from jax.experimental.pallas import tpu_sc as plsc

N = 10000
NPAD = 10240
H = 128
BLK = 512
NBLK = NPAD // BLK

NC = 2
NS = 16
EB = 128
KB = 158
EPAD = NS * KB * EB
RPT = NPAD // NS
HH = H // 2


def _ln(u, g, b):
    mu = jnp.mean(u, axis=-1, keepdims=True)
    var = jnp.mean((u - mu) ** 2, axis=-1, keepdims=True)
    return (u - mu) / jnp.sqrt(var + 1e-5) * g + b


def _store_ep(ep_ref, e, p):
    ep_ref[0, 0] = e[:, :HH]
    ep_ref[0, 1] = e[:, HH:]
    ep_ref[1, 0] = p[:, :HH]
    ep_ref[1, 1] = p[:, HH:]


def _agg_from_sw(sw_ref):
    return jnp.concatenate(
        [sw_ref[1, 0] / (sw_ref[0, 0] + 1e-16),
         sw_ref[1, 1] / (sw_ref[0, 1] + 1e-16)], axis=-1)


def _embed(xp, fc_w, fc_b, t0):
    def body(t_ref, x_ref, w_ref, b_ref, h_ref, ep_ref):
        h = jnp.maximum(
            jnp.dot(x_ref[...], w_ref[...], preferred_element_type=jnp.float32)
            + b_ref[...], 0.0)
        h_ref[...] = h
        m = h + 1e-7
        e = jnp.exp(t_ref[0, 0] * m)
        _store_ep(ep_ref, e, m * e)

    return pl.pallas_call(
        body,
        grid=(NBLK,),
        in_specs=[
            pl.BlockSpec(memory_space=pltpu.SMEM),
            pl.BlockSpec((BLK, H), lambda b: (b, 0)),
            pl.BlockSpec((H, H), lambda b: (0, 0)),
            pl.BlockSpec((1, H), lambda b: (0, 0)),
        ],
        out_specs=[
            pl.BlockSpec((BLK, H), lambda b: (b, 0)),
            pl.BlockSpec((2, 2, BLK, HH), lambda b: (0, 0, b, 0)),
        ],
        out_shape=[
            jax.ShapeDtypeStruct((NPAD, H), jnp.float32),
            jax.ShapeDtypeStruct((2, 2, NPAD, HH), jnp.float32),
        ],
    )(t0, xp, fc_w, fc_b.reshape(1, -1))


def _conv(h, sw, cp, t_next, residual):
    has_next = t_next is not None

    def body(*refs):
        if residual:
            (t_ref, h_ref, sw_ref, w1, b1, g1, be1, w2, b2, ng, nb) = refs[:11]
            outs = refs[11:]
        else:
            (t_ref, h_ref, sw_ref, w1, b1, g1, be1, w2, b2) = refs[:9]
            outs = refs[9:]
        h_out_ref = outs[0]
        agg = _agg_from_sw(sw_ref)
        h_in = h_ref[...]
        z = agg + h_in
        u = jnp.dot(z, w1[...], preferred_element_type=jnp.float32) + b1[...]
        u = jnp.maximum(_ln(u, g1[...], be1[...]), 0.0)
        v = jnp.dot(u, w2[...], preferred_element_type=jnp.float32) + b2[...]
        if residual:
            h_new = h_in + jnp.maximum(_ln(v, ng[...], nb[...]), 0.0)
        else:
            h_new = v
        h_out_ref[...] = h_new
        if has_next:
            ep_ref = outs[1]
            m = jnp.maximum(h_new, 0.0) + 1e-7
            e = jnp.exp(t_ref[0, 0] * m)
            _store_ep(ep_ref, e, m * e)

    row = lambda shp: pl.BlockSpec(shp, lambda b: (0, 0))
    in_specs = [
        pl.BlockSpec(memory_space=pltpu.SMEM),
        pl.BlockSpec((BLK, H), lambda b: (b, 0)),
        pl.BlockSpec((2, 2, BLK, HH), lambda b: (0, 0, b, 0)),
        row((H, 2 * H)), row((1, 2 * H)), row((1, 2 * H)), row((1, 2 * H)),
        row((2 * H, H)), row((1, H)),
    ]
    args = [
        jnp.ones((1, 1), jnp.float32) if t_next is None else t_next,
        h, sw,
        cp["w1"], cp["b1"].reshape(1, -1),
        cp["ln_g"].reshape(1, -1), cp["ln_b"].reshape(1, -1),
        cp["w2"], cp["b2"].reshape(1, -1),
    ]
    if residual:
        in_specs += [row((1, H)), row((1, H))]
        args += [cp["nrm_g"].reshape(1, -1), cp["nrm_b"].reshape(1, -1)]
    out_specs = [pl.BlockSpec((BLK, H), lambda b: (b, 0))]
    out_shape = [jax.ShapeDtypeStruct((NPAD, H), jnp.float32)]
    if has_next:
        out_specs.append(pl.BlockSpec((2, 2, BLK, HH), lambda b: (0, 0, b, 0)))
        out_shape.append(jax.ShapeDtypeStruct((2, 2, NPAD, HH), jnp.float32))

    res = pl.pallas_call(
        body, grid=(NBLK,),
        in_specs=in_specs, out_specs=out_specs, out_shape=out_shape,
    )(*args)
    return res if has_next else (res[0], None)


def _segsum(v4, srcb, dstb):
    mesh = plsc.VectorSubcoreMesh(core_axis_name="c", subcore_axis_name="s")

    @functools.partial(
        pl.kernel,
        out_type=jax.ShapeDtypeStruct((NC, 2, NPAD, HH), jnp.float32),
        mesh=mesh,
        compiler_params=pltpu.CompilerParams(use_tc_tiling_on_sc=False),
        scratch_types=[
            pltpu.VMEM((KB, EB), jnp.int32),
            pltpu.VMEM((KB, EB), jnp.int32),
            pltpu.VMEM((2, EB, HH), jnp.float32),
            pltpu.VMEM((EB, HH), jnp.float32),
            pltpu.VMEM_SHARED((NPAD, HH), jnp.float32),
            pltpu.SemaphoreType.DMA((2,)),
        ],
    )
    def k(v4_hbm, srcb_hbm, dstb_hbm, out_hbm, src_v, dst_v, rows_v, zb_v,
          acc, gsem):
        c = lax.axis_index("c")
        s = lax.axis_index("s")
        pltpu.sync_copy(dstb_hbm.at[s], dst_v)

        zeros16 = jnp.zeros((16,), jnp.float32)

        def zrow(i, carry):
            for j in range(HH // 16):
                zb_v[i, pl.ds(j * 16, 16)] = zeros16
            return carry

        lax.fori_loop(0, EB, zrow, 0)

        def gather_cp(j, b):
            return pltpu.make_async_copy(
                v4_hbm.at[src_v.at[j]], rows_v.at[b], gsem.at[b])

        for h in range(2):
            pltpu.sync_copy(srcb_hbm.at[c, h, s], src_v)
            for r in range(RPT // EB):
                pltpu.sync_copy(zb_v, acc.at[pl.ds(s * RPT + r * EB, EB)])
            plsc.subcore_barrier()

            gather_cp(0, 0).start()
            gather_cp(1, 1).start()

            def step(g, carry):
                for b in range(2):
                    j = g * 2 + b
                    gather_cp(j, b).wait()
                    pltpu.sync_copy(rows_v.at[b], acc.at[dst_v.at[j]],
                                    add=True)

                    @pl.when(j + 2 < KB)
                    def _():
                        gather_cp(j + 2, b).start()
                return carry

            lax.fori_loop(0, KB // 2, step, 0)
            plsc.subcore_barrier()
            pltpu.sync_copy(acc.at[pl.ds(s * RPT, RPT)],
                            out_hbm.at[c, h, pl.ds(s * RPT, RPT)])

    return k(v4, srcb, dstb)


def _attn(hs, p):
    def body(h0r, h1r, h2r, h3r, phw, phb, awr, abr, bwr, bbr, cwr, cbr,
             rwr, rbr, clw, clb, out_ref, vacc, msc, zsc):
        b = pl.program_id(0)

        @pl.when(b == 0)
        def _():
            msc[0, 0] = -1e30
            zsc[0, 0] = 0.0
            vacc[...] = jnp.zeros_like(vacc)

        hp = jnp.maximum(
            jnp.dot(h0r[...], phw[0:H], preferred_element_type=jnp.float32)
            + jnp.dot(h1r[...], phw[H:2 * H], preferred_element_type=jnp.float32)
            + jnp.dot(h2r[...], phw[2 * H:3 * H], preferred_element_type=jnp.float32)
            + jnp.dot(h3r[...], phw[3 * H:], preferred_element_type=jnp.float32)
            + phb[...], 0.0)
        a = jnp.tanh(jnp.dot(hp, awr[...], preferred_element_type=jnp.float32)
                     + abr[...])
        g = jax.nn.sigmoid(jnp.dot(hp, bwr[...], preferred_element_type=jnp.float32)
                           + bbr[...])
        sc = jnp.dot(a * g, cwr[...], preferred_element_type=jnp.float32) + cbr[...]
        rows = b * BLK + lax.broadcasted_iota(jnp.int32, (BLK, 1), 0)
        sc = jnp.where(rows < N, sc, -1e30)

        m_old = msc[0, 0]
        m_new = jnp.maximum(m_old, jnp.max(sc))
        scale = jnp.exp(m_old - m_new)
        e = jnp.exp(sc - m_new)
        zsc[0, 0] = zsc[0, 0] * scale + jnp.sum(e)
        vacc[...] = vacc[...] * scale + jnp.sum(hp * e, axis=0, keepdims=True)
        msc[0, 0] = m_new

        hpool = vacc[...] / zsc[0, 0]
        hres = jnp.maximum(
            jnp.dot(hpool, rwr[...], preferred_element_type=jnp.float32)
            + rbr[...], 0.0)
        logit = jnp.dot(hres, clw[...], preferred_element_type=jnp.float32) + clb[...]
        out_ref[...] = jax.nn.sigmoid(logit)

    D4 = 4 * H
    row = lambda shp: pl.BlockSpec(shp, lambda b: (0, 0))
    hblk = pl.BlockSpec((BLK, H), lambda b: (b, 0))
    return pl.pallas_call(
        body,
        grid=(NBLK,),
        in_specs=[hblk, hblk, hblk, hblk,
                  row((D4, D4)), row((1, D4)),
                  row((D4, D4)), row((1, D4)),
                  row((D4, D4)), row((1, D4)),
                  row((D4, 1)), row((1, 1)),
                  row((D4, D4)), row((1, D4)),
                  row((D4, 1)), row((1, 1))],
        out_specs=pl.BlockSpec((1, 1), lambda b: (0, 0)),
        out_shape=jax.ShapeDtypeStruct((1, 1), jnp.float32),
        scratch_shapes=[
            pltpu.VMEM((1, D4), jnp.float32),
            pltpu.SMEM((1, 1), jnp.float32),
            pltpu.SMEM((1, 1), jnp.float32),
        ],
    )(hs[0], hs[1], hs[2], hs[3],
      p["phi_w"], p["phi_b"].reshape(1, -1),
      p["aw"], p["ab"].reshape(1, -1),
      p["bw"], p["bb"].reshape(1, -1),
      p["cw"], p["cb"].reshape(1, 1),
      p["rho_w"], p["rho_b"].reshape(1, -1),
      p["cls_w"], p["cls_b"].reshape(1, 1))


def kernel(x, edge_index, batch, params):
    src = edge_index[0]
    dst = edge_index[1]
    pad = EPAD - src.shape[0]
    srcp = jnp.concatenate([src, jnp.full((pad,), N, jnp.int32)])
    dstp = jnp.concatenate([dst, jnp.full((pad,), N, jnp.int32)])
    srcb = (srcp[None, None] +
            (jnp.arange(4, dtype=jnp.int32) * NPAD).reshape(2, 2, 1)
            ).reshape(NC, 2, NS, KB, EB)
    dstb = dstp.reshape(NS, KB, EB)
    xp = jnp.pad(x, ((0, NPAD - N), (0, 0)))

    cs = params["convs"]
    t = [c["t"].reshape(1, 1) for c in cs]

    h0, ep0 = _embed(xp, params["fc_w"], params["fc_b"], t[0])
    sw0 = _segsum(ep0.reshape(4 * NPAD, HH), srcb, dstb)
    h1, ep1 = _conv(h0, sw0, cs[0], t[1], residual=False)
    sw1 = _segsum(ep1.reshape(4 * NPAD, HH), srcb, dstb)
    h2, ep2 = _conv(h1, sw1, cs[1], t[2], residual=True)
    sw2 = _segsum(ep2.reshape(4 * NPAD, HH), srcb, dstb)
    h3, _ = _conv(h2, sw2, cs[2], None, residual=True)
    return _attn((h0, h1, h2, h3), params)
